# baseline (speedup 1.0000x reference)
import functools

import jax
import jax.numpy as jnp
from jax import lax
from jax.experimental import pallas as pl
from jax.experimental.pallas import tpu as pltpu
from jax.experimental.pallas import tpu_sc as plsc

_D = 32
_B = 16384
_K = 10
_NC = 2
_NS = 16
_NW = _NC * _NS
_BPW = _B // _NW
_CB = 64
_NCHUNK = _BPW // _CB
_G = _CB // 16

_sc_mesh = plsc.VectorSubcoreMesh(core_axis_name="c", subcore_axis_name="s")


@functools.partial(
    pl.kernel,
    out_type=(
        jax.ShapeDtypeStruct((_B,), jnp.float32),
        jax.ShapeDtypeStruct((_B * _K,), jnp.float32),
    ),
    mesh=_sc_mesh,
    scratch_types=(
        pltpu.VMEM((_CB,), jnp.int32),
        pltpu.VMEM((_CB,), jnp.int32),
        pltpu.VMEM((_CB,), jnp.int32),
        pltpu.VMEM((_CB,), jnp.int32),
        pltpu.VMEM((_CB * _K,), jnp.int32),
        pltpu.VMEM((_CB * _K,), jnp.int32),
        pltpu.VMEM((_CB, 4 * _D), jnp.float32),
        pltpu.VMEM((_CB, 4 * _D), jnp.float32),
        pltpu.VMEM((_CB * _K, 4 * _D), jnp.float32),
        pltpu.VMEM((_BPW,), jnp.float32),
        pltpu.VMEM((_BPW * _K,), jnp.float32),
        pltpu.SemaphoreType.DMA,
    ),
    compiler_params=pltpu.CompilerParams(needs_layout_passes=False),
)
def _sc_scores(tq, tcb, cq, ccb, nq, ncb, ttab, ctab, pos_out, neg_out,
               tq_v, tcb_v, cq_v, ccb_v, nq_v, ncb_v,
               trow_v, crow_v, nrow_v, pos_v, neg_v, sem):
    wid = lax.axis_index("s") * _NC + lax.axis_index("c")
    base = wid * _BPW
    ji = lax.iota(jnp.int32, 16)

    def chunk_body(c, carry):
        boff = base + c * _CB
        pltpu.sync_copy(tq.at[pl.ds(boff, _CB)], tq_v)
        pltpu.sync_copy(tcb.at[pl.ds(boff, _CB)], tcb_v)
        pltpu.sync_copy(cq.at[pl.ds(boff, _CB)], cq_v)
        pltpu.sync_copy(ccb.at[pl.ds(boff, _CB)], ccb_v)
        pltpu.sync_copy(nq.at[pl.ds(boff * _K, _CB * _K)], nq_v)
        pltpu.sync_copy(ncb.at[pl.ds(boff * _K, _CB * _K)], ncb_v)
        pltpu.async_copy(ttab.at[tq_v], trow_v, sem).wait()
        pltpu.async_copy(ctab.at[cq_v], crow_v, sem).wait()
        pltpu.async_copy(ctab.at[nq_v], nrow_v, sem).wait()
        for g in range(_G):
            rows = ji + g * 16
            tcbv = tcb_v[pl.ds(g * 16, 16)]
            ccbv = ccb_v[pl.ds(g * 16, 16)]
            nrows = []
            ncbvs = []
            for k in range(_K):
                nr = ji * _K + (k + g * 16 * _K)
                nrows.append(nr)
                ncbvs.append(plsc.load_gather(ncb_v, [nr]))

            def dot_body(d, accs):
                ap = accs[0]
                an = accs[1]
                td = plsc.load_gather(trow_v, [rows, tcbv + d])
                cd = plsc.load_gather(crow_v, [rows, ccbv + d])
                ap = ap + td * cd
                an_new = tuple(
                    an[k] + td * plsc.load_gather(nrow_v, [nrows[k], ncbvs[k] + d])
                    for k in range(_K))
                return (ap, an_new)

            zero = jnp.zeros((16,), jnp.float32)
            acc_pos, acc_neg = lax.fori_loop(
                0, _D, dot_body, (zero, (zero,) * _K))
            loc = c * _CB + g * 16
            pos_v[pl.ds(loc, 16)] = acc_pos
            for k in range(_K):
                plsc.store_scatter(neg_v, [(loc + ji) * _K + k], acc_neg[k])
        return carry

    lax.fori_loop(0, _NCHUNK, chunk_body, 0)
    pltpu.sync_copy(pos_v, pos_out.at[pl.ds(base, _BPW)])
    pltpu.sync_copy(neg_v, neg_out.at[pl.ds(base * _K, _BPW * _K)])


_VB = 512
_NVB = (1000000 + _VB - 1) // _VB
_PROWS = _NVB * 128


def _tc_pack_body(t_ref, c_ref, to_ref, co_ref):
    for src, dst in ((t_ref, to_ref), (c_ref, co_ref)):
        x = src[...]
        dst[...] = jnp.concatenate(
            [x[:, jj * 128:(jj + 1) * 128].T for jj in range(4)], axis=1)


_tc_pack = pl.pallas_call(
    _tc_pack_body,
    grid=(_NVB,),
    in_specs=[
        pl.BlockSpec((_D, _VB), lambda i: (0, i)),
        pl.BlockSpec((_D, _VB), lambda i: (0, i)),
    ],
    out_specs=[
        pl.BlockSpec((128, 4 * _D), lambda i: (i, 0)),
        pl.BlockSpec((128, 4 * _D), lambda i: (i, 0)),
    ],
    out_shape=[
        jax.ShapeDtypeStruct((_PROWS, 4 * _D), jnp.float32),
        jax.ShapeDtypeStruct((_PROWS, 4 * _D), jnp.float32),
    ],
)


def _tc_loss_body(p_ref, n_ref, o_ref):
    s = jnp.sum(jax.nn.log_sigmoid(p_ref[...]))
    s = s + jnp.sum(jax.nn.log_sigmoid(-n_ref[...]))
    o_ref[...] = jnp.zeros_like(o_ref) - s


_tc_loss = pl.pallas_call(
    _tc_loss_body,
    out_shape=jax.ShapeDtypeStruct((1, 1), jnp.float32),
)


def kernel(target_idx, context_idx, neg_idx, emb_target_table, emb_context_table):
    tix = target_idx.astype(jnp.int32)
    cix = context_idx.astype(jnp.int32)
    nix = neg_idx.astype(jnp.int32).reshape(-1)
    ttab_packed, ctab_packed = _tc_pack(emb_target_table.T, emb_context_table.T)

    def prow(ix):
        return (ix >> 9) * 128 + (ix & 127)

    def pcol(ix):
        return ((ix >> 7) & 3) * _D

    pos, neg = _sc_scores(
        prow(tix), pcol(tix),
        prow(cix), pcol(cix),
        prow(nix), pcol(nix),
        ttab_packed,
        ctab_packed,
    )
    out = _tc_loss(pos.reshape(128, 128), neg.reshape(1280, 128))
    return out.reshape(())


# speedup vs baseline: 3.3057x; 3.3057x over previous
import functools

import jax
import jax.numpy as jnp
from jax import lax
from jax.experimental import pallas as pl
from jax.experimental.pallas import tpu as pltpu
from jax.experimental.pallas import tpu_sc as plsc

_D = 32
_B = 16384
_K = 10
_NC = 2
_NS = 16
_NW = _NC * _NS
_BPW = _B // _NW
_CB = 64
_NCHUNK = _BPW // _CB
_G = _CB // 16

_VB = 4096
_NVB = (1000000 + _VB - 1) // _VB
_PROWS = _NVB * (_VB // 4)


_sc_mesh = plsc.VectorSubcoreMesh(core_axis_name="c", subcore_axis_name="s")


def _tc_pack_body(t_ref, c_ref, to_ref, co_ref):
    er = lax.broadcasted_iota(jnp.int32, (128, 128), 0)
    ec = lax.broadcasted_iota(jnp.int32, (128, 128), 1)
    eye = (er == ec).astype(jnp.float32)
    dn = (((0,), (0,)), ((), ()))
    for src, dst in ((t_ref, to_ref), (c_ref, co_ref)):
        x = src[...]
        for b in range(_VB // 512):
            xcat = jnp.concatenate(
                [x[:, b * 512 + jj * 128:b * 512 + (jj + 1) * 128]
                 for jj in range(4)], axis=0)
            dst[b * 128:(b + 1) * 128, :] = lax.dot_general(
                xcat, eye, dn, preferred_element_type=jnp.float32)


_tc_pack = pl.pallas_call(
    _tc_pack_body,
    grid=(_NVB,),
    in_specs=[
        pl.BlockSpec((_D, _VB), lambda i: (0, i)),
        pl.BlockSpec((_D, _VB), lambda i: (0, i)),
    ],
    out_specs=[
        pl.BlockSpec((_VB // 4, 4 * _D), lambda i: (i, 0)),
        pl.BlockSpec((_VB // 4, 4 * _D), lambda i: (i, 0)),
    ],
    out_shape=[
        jax.ShapeDtypeStruct((_PROWS, 4 * _D), jnp.float32),
        jax.ShapeDtypeStruct((_PROWS, 4 * _D), jnp.float32),
    ],
)


@functools.partial(
    pl.kernel,
    out_type=(
        jax.ShapeDtypeStruct((_B,), jnp.float32),
        jax.ShapeDtypeStruct((_B * _K,), jnp.float32),
    ),
    mesh=_sc_mesh,
    scratch_types=(
        pltpu.VMEM((_CB,), jnp.int32),
        pltpu.VMEM((_CB,), jnp.int32),
        pltpu.VMEM((_CB * _K,), jnp.int32),
        pltpu.VMEM((_CB, _D), jnp.float32),
        pltpu.VMEM((_CB, _D), jnp.float32),
        pltpu.VMEM((_CB * _K, _D), jnp.float32),
        pltpu.VMEM((_BPW,), jnp.float32),
        pltpu.VMEM((_BPW * _K,), jnp.float32),
        pltpu.SemaphoreType.DMA,
    ),
    compiler_params=pltpu.CompilerParams(
        needs_layout_passes=False, use_tc_tiling_on_sc=False),
)
def _sc_scores(tq, cq, nq, ttab, ctab, pos_out, neg_out,
               tq_v, cq_v, nq_v,
               trow_v, crow_v, nrow_v, pos_v, neg_v, sem):
    wid = lax.axis_index("s") * _NC + lax.axis_index("c")
    base = wid * _BPW
    ji = lax.iota(jnp.int32, 16)

    def chunk_body(c, carry):
        boff = base + c * _CB
        pltpu.sync_copy(tq.at[pl.ds(boff, _CB)], tq_v)
        pltpu.sync_copy(cq.at[pl.ds(boff, _CB)], cq_v)
        pltpu.sync_copy(nq.at[pl.ds(boff * _K, _CB * _K)], nq_v)
        pltpu.async_copy(ttab.at[tq_v], trow_v, sem).wait()
        pltpu.async_copy(ctab.at[cq_v], crow_v, sem).wait()
        pltpu.async_copy(ctab.at[nq_v], nrow_v, sem).wait()
        for g in range(_G):
            rows = ji + g * 16
            nrows = [ji * _K + (k + g * 16 * _K) for k in range(_K)]

            def dot_body(d, accs):
                ap = accs[0]
                an = accs[1]
                dv = jnp.full((16,), d, jnp.int32)
                td = plsc.load_gather(trow_v, [rows, dv])
                cd = plsc.load_gather(crow_v, [rows, dv])
                ap = ap + td * cd
                an_new = tuple(
                    an[k] + td * plsc.load_gather(nrow_v, [nrows[k], dv])
                    for k in range(_K))
                return (ap, an_new)

            zero = jnp.zeros((16,), jnp.float32)
            acc_pos, acc_neg = lax.fori_loop(
                0, _D, dot_body, (zero, (zero,) * _K))
            loc = c * _CB + g * 16
            pos_v[pl.ds(loc, 16)] = acc_pos
            for k in range(_K):
                plsc.store_scatter(neg_v, [(loc + ji) * _K + k], acc_neg[k])
        return carry

    lax.fori_loop(0, _NCHUNK, chunk_body, 0)
    pltpu.sync_copy(pos_v, pos_out.at[pl.ds(base, _BPW)])
    pltpu.sync_copy(neg_v, neg_out.at[pl.ds(base * _K, _BPW * _K)])


def _tc_loss_body(p_ref, n_ref, o_ref):
    s = jnp.sum(jax.nn.log_sigmoid(p_ref[...]))
    s = s + jnp.sum(jax.nn.log_sigmoid(-n_ref[...]))
    o_ref[...] = jnp.zeros_like(o_ref) - s


_tc_loss = pl.pallas_call(
    _tc_loss_body,
    out_shape=jax.ShapeDtypeStruct((1, 1), jnp.float32),
)


def kernel(target_idx, context_idx, neg_idx, emb_target_table, emb_context_table):
    tix = target_idx.astype(jnp.int32)
    cix = context_idx.astype(jnp.int32)
    nix = neg_idx.astype(jnp.int32).reshape(-1)
    ttab_packed, ctab_packed = _tc_pack(emb_target_table.T, emb_context_table.T)

    def prow(ix):
        return (ix >> 9) * 512 + (ix & 127) * 4 + ((ix >> 7) & 3)

    pos, neg = _sc_scores(
        prow(tix), prow(cix), prow(nix),
        ttab_packed.reshape(4 * _PROWS, _D),
        ctab_packed.reshape(4 * _PROWS, _D),
    )
    out = _tc_loss(pos.reshape(128, 128), neg.reshape(1280, 128))
    return out.reshape(())


# speedup vs baseline: 3.5195x; 1.0647x over previous
import functools

import jax
import jax.numpy as jnp
from jax import lax
from jax.experimental import pallas as pl
from jax.experimental.pallas import tpu as pltpu
from jax.experimental.pallas import tpu_sc as plsc

_D = 32
_B = 16384
_K = 10
_NC = 2
_NS = 16
_NW = _NC * _NS
_BPW = _B // _NW
_CB = 64
_NCHUNK = _BPW // _CB
_G = _CB // 16

_VB = 4096
_NVB = (1000000 + _VB - 1) // _VB
_PROWS = _NVB * (_VB // 4)


_sc_mesh = plsc.VectorSubcoreMesh(core_axis_name="c", subcore_axis_name="s")


def _tc_pack_body(t_ref, c_ref, to_ref, co_ref):
    er = lax.broadcasted_iota(jnp.int32, (128, 128), 0)
    ec = lax.broadcasted_iota(jnp.int32, (128, 128), 1)
    eye = (er == ec).astype(jnp.float32)
    dn = (((0,), (0,)), ((), ()))
    for src, dst in ((t_ref, to_ref), (c_ref, co_ref)):
        x = src[...]
        for b in range(_VB // 512):
            xcat = jnp.concatenate(
                [x[:, b * 512 + jj * 128:b * 512 + (jj + 1) * 128]
                 for jj in range(4)], axis=0)
            dst[b * 128:(b + 1) * 128, :] = lax.dot_general(
                xcat, eye, dn, preferred_element_type=jnp.float32)


_tc_pack = pl.pallas_call(
    _tc_pack_body,
    grid=(_NVB,),
    in_specs=[
        pl.BlockSpec((_D, _VB), lambda i: (0, i)),
        pl.BlockSpec((_D, _VB), lambda i: (0, i)),
    ],
    out_specs=[
        pl.BlockSpec((_VB // 4, 4 * _D), lambda i: (i, 0)),
        pl.BlockSpec((_VB // 4, 4 * _D), lambda i: (i, 0)),
    ],
    out_shape=[
        jax.ShapeDtypeStruct((_PROWS, 4 * _D), jnp.float32),
        jax.ShapeDtypeStruct((_PROWS, 4 * _D), jnp.float32),
    ],
)


@functools.partial(
    pl.kernel,
    out_type=(
        jax.ShapeDtypeStruct((_B,), jnp.float32),
        jax.ShapeDtypeStruct((_B * _K,), jnp.float32),
    ),
    mesh=_sc_mesh,
    scratch_types=(
        pltpu.VMEM((_BPW,), jnp.int32),
        pltpu.VMEM((_BPW,), jnp.int32),
        pltpu.VMEM((_BPW * _K,), jnp.int32),
        pltpu.VMEM((2, _CB, _D), jnp.float32),
        pltpu.VMEM((2, _CB, _D), jnp.float32),
        pltpu.VMEM((2, _CB * _K, _D), jnp.float32),
        pltpu.VMEM((_BPW,), jnp.float32),
        pltpu.VMEM((_BPW * _K,), jnp.float32),
        pltpu.SemaphoreType.DMA,
        pltpu.SemaphoreType.DMA,
    ),
    compiler_params=pltpu.CompilerParams(
        needs_layout_passes=False, use_tc_tiling_on_sc=False),
)
def _sc_scores(tq, cq, nq, ttab, ctab, pos_out, neg_out,
               tq_v, cq_v, nq_v,
               trow_v, crow_v, nrow_v, pos_v, neg_v, sem0, sem1):
    wid = lax.axis_index("s") * _NC + lax.axis_index("c")
    base = wid * _BPW
    ji = lax.iota(jnp.int32, 16)
    pltpu.sync_copy(tq.at[pl.ds(base, _BPW)], tq_v)
    pltpu.sync_copy(cq.at[pl.ds(base, _BPW)], cq_v)
    pltpu.sync_copy(nq.at[pl.ds(base * _K, _BPW * _K)], nq_v)
    sems = (sem0, sem1)

    def issue(c):
        buf = c % 2
        sem = sems[buf]
        return (
            pltpu.async_copy(
                ttab.at[tq_v.at[pl.ds(c * _CB, _CB)]], trow_v.at[buf], sem),
            pltpu.async_copy(
                ctab.at[cq_v.at[pl.ds(c * _CB, _CB)]], crow_v.at[buf], sem),
            pltpu.async_copy(
                ctab.at[nq_v.at[pl.ds(c * _CB * _K, _CB * _K)]],
                nrow_v.at[buf], sem),
        )

    descs = issue(0)
    for c in range(_NCHUNK):
        nxt = issue(c + 1) if c + 1 < _NCHUNK else ()
        for dsc in descs:
            dsc.wait()
        buf = c % 2
        for g in range(_G):
            rows = ji + g * 16
            nrows = [ji * _K + (k + g * 16 * _K) for k in range(_K)]

            def dot_body(d, accs):
                ap = accs[0]
                an = accs[1]
                dv = jnp.full((16,), d, jnp.int32)
                td = plsc.load_gather(trow_v.at[buf], [rows, dv])
                cd = plsc.load_gather(crow_v.at[buf], [rows, dv])
                ap = ap + td * cd
                an_new = tuple(
                    an[k] + td * plsc.load_gather(
                        nrow_v.at[buf], [nrows[k], dv])
                    for k in range(_K))
                return (ap, an_new)

            zero = jnp.zeros((16,), jnp.float32)
            acc_pos, acc_neg = lax.fori_loop(
                0, _D, dot_body, (zero, (zero,) * _K))
            loc = c * _CB + g * 16
            pos_v[pl.ds(loc, 16)] = acc_pos
            for k in range(_K):
                plsc.store_scatter(neg_v, [(loc + ji) * _K + k], acc_neg[k])
        descs = nxt
    pltpu.sync_copy(pos_v, pos_out.at[pl.ds(base, _BPW)])
    pltpu.sync_copy(neg_v, neg_out.at[pl.ds(base * _K, _BPW * _K)])


def _tc_loss_body(p_ref, n_ref, o_ref):
    s = jnp.sum(jax.nn.log_sigmoid(p_ref[...]))
    s = s + jnp.sum(jax.nn.log_sigmoid(-n_ref[...]))
    o_ref[...] = jnp.zeros_like(o_ref) - s


_tc_loss = pl.pallas_call(
    _tc_loss_body,
    out_shape=jax.ShapeDtypeStruct((1, 1), jnp.float32),
)


def kernel(target_idx, context_idx, neg_idx, emb_target_table, emb_context_table):
    tix = target_idx.astype(jnp.int32)
    cix = context_idx.astype(jnp.int32)
    nix = neg_idx.astype(jnp.int32).reshape(-1)
    ttab_packed, ctab_packed = _tc_pack(emb_target_table.T, emb_context_table.T)

    def prow(ix):
        return (ix >> 9) * 512 + (ix & 127) * 4 + ((ix >> 7) & 3)

    pos, neg = _sc_scores(
        prow(tix), prow(cix), prow(nix),
        ttab_packed.reshape(4 * _PROWS, _D),
        ctab_packed.reshape(4 * _PROWS, _D),
    )
    out = _tc_loss(pos.reshape(128, 128), neg.reshape(1280, 128))
    return out.reshape(())
